# baseline (device time: 147571 ns/iter reference)
import jax
import jax.numpy as jnp
from jax import lax
from jax.experimental import pallas as pl
from jax.experimental.pallas import tpu as pltpu

B = 32
H = 16
D = 128
BS = 32
NB = 256
P_LOCAL = 256
P_DEV = 128
KB_PAGES = 16
KB_TOK = KB_PAGES * BS
N_KB = P_DEV // KB_PAGES
G = 4
HG = H // G
GB = HG * B
GD = HG * D
NEG = -1e30
SCALE = D ** -0.5
MESH = pl.DeviceIdType.MESH


def _attn_body(xref, qbd_ref, k_ref, v_ref, bt_ref, lens_ref,
               acc_ref, m_ref, l_ref, bias_ref, m_sc, l_sc):
    kb = pl.program_id(0)
    my_y = lax.axis_index("y")

    @pl.when(kb == 0)
    def _init():
        m_sc[...] = jnp.full((G, GB, 1), NEG, jnp.float32)
        l_sc[...] = jnp.zeros((G, GB, 1), jnp.float32)
        acc_ref[...] = jnp.zeros((H, B, D), jnp.float32)
        bt = bt_ref[...]
        lens = lens_ref[...]
        slot = lax.broadcasted_iota(jnp.int32, (1, 1, NB), 2)
        valid = slot < lens[None, :, :]
        base = my_y * P_LOCAL + xref[0] * KB_PAGES
        CH = 32
        for c in range(P_DEV // CH):
            pages = base + c * CH + lax.broadcasted_iota(
                jnp.int32, (CH, 1, 1), 0)
            eq = bt[None, :, :] == pages
            cnt = jnp.sum(jnp.where(eq & valid, 1.0, 0.0), axis=2)
            bias_ref[c * CH:(c + 1) * CH, :] = jnp.where(
                cnt > 0.5, jnp.log(cnt), NEG)

    bias_blk = bias_ref[pl.ds(kb * KB_PAGES, KB_PAGES), :]
    rows = lax.broadcasted_iota(jnp.int32, (KB_PAGES, KB_TOK), 0)
    cols = lax.broadcasted_iota(jnp.int32, (KB_PAGES, KB_TOK), 1)
    expand = jnp.where(cols // BS == rows, 1.0, 0.0)
    bias_tok = lax.dot_general(
        bias_blk, expand, (((0,), (0,)), ((), ())),
        preferred_element_type=jnp.float32)
    bias_g = jnp.tile(bias_tok, (HG, 1))

    for g in range(G):
        kg = k_ref[:, g * GD:(g + 1) * GD].astype(jnp.bfloat16)
        vg = v_ref[:, g * GD:(g + 1) * GD].astype(jnp.bfloat16)
        s = lax.dot_general(
            qbd_ref[g], kg, (((1,), (1,)), ((), ())),
            preferred_element_type=jnp.float32)
        s = s + bias_g
        m_old = m_sc[g]
        m_new = jnp.maximum(m_old, jnp.max(s, axis=1, keepdims=True))
        p = jnp.exp(s - m_new)
        corr = jnp.exp(m_old - m_new)
        m_sc[g] = m_new
        l_sc[g] = l_sc[g] * corr + jnp.sum(p, axis=1, keepdims=True)
        o = lax.dot_general(
            p.astype(jnp.bfloat16), vg, (((1,), (0,)), ((), ())),
            preferred_element_type=jnp.float32)
        for a in range(HG):
            acc_ref[g * HG + a] = (
                acc_ref[g * HG + a] * corr[a * B:(a + 1) * B]
                + o[a * B:(a + 1) * B, a * D:(a + 1) * D])

    @pl.when(kb == N_KB - 1)
    def _finish():
        for g in range(G):
            for a in range(HG):
                m_ref[g * HG + a] = m_sc[g, a * B:(a + 1) * B]
                l_ref[g * HG + a] = l_sc[g, a * B:(a + 1) * B]


def _partial(xarr, qbd, k, v, bt, lens2):
    grid_spec = pltpu.PrefetchScalarGridSpec(
        num_scalar_prefetch=1,
        grid=(N_KB,),
        in_specs=[
            pl.BlockSpec((G, GB, GD), lambda kb, xr: (0, 0, 0)),
            pl.BlockSpec((KB_TOK, H * D), lambda kb, xr: (xr[0] + kb, 0)),
            pl.BlockSpec((KB_TOK, H * D), lambda kb, xr: (xr[0] + kb, 0)),
            pl.BlockSpec((B, NB), lambda kb, xr: (0, 0)),
            pl.BlockSpec((B, 1), lambda kb, xr: (0, 0)),
        ],
        out_specs=[
            pl.BlockSpec((H, B, D), lambda kb, xr: (0, 0, 0)),
            pl.BlockSpec((H, B, 1), lambda kb, xr: (0, 0, 0)),
            pl.BlockSpec((H, B, 1), lambda kb, xr: (0, 0, 0)),
        ],
        scratch_shapes=[
            pltpu.VMEM((P_DEV, B), jnp.float32),
            pltpu.VMEM((G, GB, 1), jnp.float32),
            pltpu.VMEM((G, GB, 1), jnp.float32),
        ],
    )
    return pl.pallas_call(
        _attn_body,
        grid_spec=grid_spec,
        out_shape=[
            jax.ShapeDtypeStruct((H, B, D), jnp.float32),
            jax.ShapeDtypeStruct((H, B, 1), jnp.float32),
            jax.ShapeDtypeStruct((H, B, 1), jnp.float32),
        ],
        compiler_params=pltpu.CompilerParams(
            dimension_semantics=("arbitrary",)),
    )(xarr, qbd, k, v, bt, lens2)


def _combine_body(acc_ref, m_ref, l_ref, out_ref,
                  r_acc, r_m, r_l, s2_acc, s2_m, s2_l,
                  r2_acc, r2_m, r2_l, send_sems, recv_sems):
    my_x = lax.axis_index("x")
    my_y = lax.axis_index("y")
    y_peer = (my_x, 1 - my_y)
    x_peer = (1 - my_x, my_y)

    barrier = pltpu.get_barrier_semaphore()
    for nbr in (y_peer, x_peer):
        pl.semaphore_signal(barrier, inc=1, device_id=nbr,
                            device_id_type=MESH)
    pl.semaphore_wait(barrier, 2)

    round1 = []
    for i, (src, dst) in enumerate(
            ((acc_ref, r_acc), (m_ref, r_m), (l_ref, r_l))):
        rdma = pltpu.make_async_remote_copy(
            src_ref=src, dst_ref=dst,
            send_sem=send_sems.at[i], recv_sem=recv_sems.at[i],
            device_id=y_peer, device_id_type=MESH)
        rdma.start()
        round1.append(rdma)
    for rdma in round1:
        rdma.wait()

    m, l, acc = m_ref[...], l_ref[...], acc_ref[...]
    mr, lr, ar = r_m[...], r_l[...], r_acc[...]
    mt = jnp.maximum(m, mr)
    a = jnp.exp(m - mt)
    b = jnp.exp(mr - mt)
    s2_m[...] = mt
    s2_l[...] = a * l + b * lr
    s2_acc[...] = a * acc + b * ar

    round2 = []
    for i, (src, dst) in enumerate(
            ((s2_acc, r2_acc), (s2_m, r2_m), (s2_l, r2_l))):
        rdma = pltpu.make_async_remote_copy(
            src_ref=src, dst_ref=dst,
            send_sem=send_sems.at[3 + i], recv_sem=recv_sems.at[3 + i],
            device_id=x_peer, device_id_type=MESH)
        rdma.start()
        round2.append(rdma)
    for rdma in round2:
        rdma.wait()

    m, l, acc = s2_m[...], s2_l[...], s2_acc[...]
    mr, lr, ar = r2_m[...], r2_l[...], r2_acc[...]
    mt = jnp.maximum(m, mr)
    a = jnp.exp(m - mt)
    b = jnp.exp(mr - mt)
    lt = a * l + b * lr
    o = (a * acc + b * ar) / lt
    for h in range(H):
        out_ref[:, 0, h, :] = o[h]


def _combine(acc, m, l):
    return pl.pallas_call(
        _combine_body,
        in_specs=[pl.BlockSpec(memory_space=pltpu.VMEM)] * 3,
        out_specs=pl.BlockSpec(memory_space=pltpu.VMEM),
        out_shape=jax.ShapeDtypeStruct((B, 1, H, D), jnp.float32),
        scratch_shapes=[
            pltpu.VMEM((H, B, D), jnp.float32),
            pltpu.VMEM((H, B, 1), jnp.float32),
            pltpu.VMEM((H, B, 1), jnp.float32),
            pltpu.VMEM((H, B, D), jnp.float32),
            pltpu.VMEM((H, B, 1), jnp.float32),
            pltpu.VMEM((H, B, 1), jnp.float32),
            pltpu.VMEM((H, B, D), jnp.float32),
            pltpu.VMEM((H, B, 1), jnp.float32),
            pltpu.VMEM((H, B, 1), jnp.float32),
            pltpu.SemaphoreType.DMA((6,)),
            pltpu.SemaphoreType.DMA((6,)),
        ],
        compiler_params=pltpu.CompilerParams(collective_id=0),
    )(acc, m, l)


def kernel(Q, K, V, bt, lens):
    my_x = lax.axis_index("x")
    q = jnp.transpose(Q.reshape(B, H, D) * SCALE, (1, 0, 2))
    qg = q.reshape(G, HG, B, D)
    eye = jnp.eye(HG, dtype=q.dtype)
    qbd = (qg[:, :, :, None, :] * eye[None, :, None, :, None]).reshape(
        G, GB, GD).astype(jnp.bfloat16)
    k = K.reshape(P_LOCAL * BS, H * D)
    v = V.reshape(P_LOCAL * BS, H * D)
    lens2 = lens.reshape(B, 1)
    xarr = jnp.full((1,), my_x * N_KB, jnp.int32)
    acc, m, l = _partial(xarr, qbd, k, v, bt, lens2)
    return _combine(acc, m, l)


# device time: 85934 ns/iter; 1.7173x vs baseline; 1.7173x over previous
import jax
import jax.numpy as jnp
from jax import lax
from jax.experimental import pallas as pl
from jax.experimental.pallas import tpu as pltpu

B = 32
H = 16
D = 128
BS = 32
NB = 256
P_LOCAL = 256
P_DEV = 128
KB_PAGES = 32
KB_TOK = KB_PAGES * BS
N_KB = P_DEV // KB_PAGES
NEG = -1e30
SCALE = D ** -0.5
MESH = pl.DeviceIdType.MESH


TOTAL_STEPS = H * N_KB


def _attn_body(xref, q_ref, k_hbm, v_hbm, bt_ref, lens_ref,
               acc_ref, m_ref, l_ref,
               bias_ref, kbuf, vbuf, ksems, vsems):
    step = pl.program_id(0)
    kb = step % N_KB
    my_y = lax.axis_index("y")

    def start_dma(s, slot):
        hh = s // N_KB
        kk = s % N_KB
        row0 = (xref[0] + kk) * KB_TOK
        pltpu.make_async_copy(
            k_hbm.at[pl.ds(row0, KB_TOK), hh], kbuf.at[slot],
            ksems.at[slot]).start()
        pltpu.make_async_copy(
            v_hbm.at[pl.ds(row0, KB_TOK), hh], vbuf.at[slot],
            vsems.at[slot]).start()

    @pl.when(step == 0)
    def _prologue():
        start_dma(0, 0)

    @pl.when(step + 1 < TOTAL_STEPS)
    def _prefetch():
        start_dma(step + 1, (step + 1) % 2)

    @pl.when(step == 0)
    def _bias():
        bt = bt_ref[...]
        lens = lens_ref[...]
        slot = lax.broadcasted_iota(jnp.int32, (1, 1, NB), 2)
        valid = slot < lens[None, :, :]
        base = my_y * P_LOCAL + xref[0] * KB_PAGES
        CH = 32
        for c in range(P_DEV // CH):
            pages = base + c * CH + lax.broadcasted_iota(
                jnp.int32, (CH, 1, 1), 0)
            eq = bt[None, :, :] == pages
            cnt = jnp.sum(jnp.where(eq & valid, 1.0, 0.0), axis=2)
            bias_ref[c * CH:(c + 1) * CH, :] = jnp.where(
                cnt > 0.5, jnp.log(cnt), NEG)

    @pl.when(kb == 0)
    def _init():
        m_ref[...] = jnp.full((1, B, 1), NEG, jnp.float32)
        l_ref[...] = jnp.zeros((1, B, 1), jnp.float32)
        acc_ref[...] = jnp.zeros((1, B, D), jnp.float32)

    bias_blk = bias_ref[pl.ds(kb * KB_PAGES, KB_PAGES), :]
    rows = lax.broadcasted_iota(jnp.int32, (KB_PAGES, KB_TOK), 0)
    cols = lax.broadcasted_iota(jnp.int32, (KB_PAGES, KB_TOK), 1)
    expand = jnp.where(cols // BS == rows, 1.0, 0.0)
    bias_tok = lax.dot_general(
        bias_blk, expand, (((0,), (0,)), ((), ())),
        preferred_element_type=jnp.float32)

    slot = step % 2
    pltpu.make_async_copy(
        k_hbm.at[pl.ds(0, KB_TOK), 0], kbuf.at[slot], ksems.at[slot]).wait()
    pltpu.make_async_copy(
        v_hbm.at[pl.ds(0, KB_TOK), 0], vbuf.at[slot], vsems.at[slot]).wait()

    qh = q_ref[0]
    kh = kbuf[slot].astype(jnp.bfloat16)
    vh = vbuf[slot].astype(jnp.bfloat16)
    s = lax.dot_general(
        qh, kh, (((1,), (1,)), ((), ())),
        preferred_element_type=jnp.float32)
    s = s + bias_tok
    m_old = m_ref[0]
    m_new = jnp.maximum(m_old, jnp.max(s, axis=1, keepdims=True))
    p = jnp.exp(s - m_new)
    corr = jnp.exp(m_old - m_new)
    m_ref[0] = m_new
    l_ref[0] = l_ref[0] * corr + jnp.sum(p, axis=1, keepdims=True)
    pv = lax.dot_general(
        p.astype(jnp.bfloat16), vh, (((1,), (0,)), ((), ())),
        preferred_element_type=jnp.float32)
    acc_ref[0] = acc_ref[0] * corr + pv


def _partial(xarr, q, k, v, bt, lens2):
    grid_spec = pltpu.PrefetchScalarGridSpec(
        num_scalar_prefetch=1,
        grid=(TOTAL_STEPS,),
        in_specs=[
            pl.BlockSpec((1, B, D), lambda i, xr: (i // N_KB, 0, 0)),
            pl.BlockSpec(memory_space=pl.ANY),
            pl.BlockSpec(memory_space=pl.ANY),
            pl.BlockSpec((B, NB), lambda i, xr: (0, 0)),
            pl.BlockSpec((B, 1), lambda i, xr: (0, 0)),
        ],
        out_specs=[
            pl.BlockSpec((1, B, D), lambda i, xr: (i // N_KB, 0, 0)),
            pl.BlockSpec((1, B, 1), lambda i, xr: (i // N_KB, 0, 0)),
            pl.BlockSpec((1, B, 1), lambda i, xr: (i // N_KB, 0, 0)),
        ],
        scratch_shapes=[
            pltpu.VMEM((P_DEV, B), jnp.float32),
            pltpu.VMEM((2, KB_TOK, D), jnp.float32),
            pltpu.VMEM((2, KB_TOK, D), jnp.float32),
            pltpu.SemaphoreType.DMA((2,)),
            pltpu.SemaphoreType.DMA((2,)),
        ],
    )
    return pl.pallas_call(
        _attn_body,
        grid_spec=grid_spec,
        out_shape=[
            jax.ShapeDtypeStruct((H, B, D), jnp.float32),
            jax.ShapeDtypeStruct((H, B, 1), jnp.float32),
            jax.ShapeDtypeStruct((H, B, 1), jnp.float32),
        ],
        compiler_params=pltpu.CompilerParams(
            dimension_semantics=("arbitrary",)),
    )(xarr, q, k, v, bt, lens2)


def _combine_body(acc_ref, m_ref, l_ref, out_ref,
                  r_acc, r_m, r_l, send_sems, recv_sems):
    my_x = lax.axis_index("x")
    my_y = lax.axis_index("y")
    peers = (
        (my_x, 1 - my_y),
        (1 - my_x, my_y),
        (1 - my_x, 1 - my_y),
    )

    barrier = pltpu.get_barrier_semaphore()
    for nbr in peers:
        pl.semaphore_signal(barrier, inc=1, device_id=nbr,
                            device_id_type=MESH)
    pl.semaphore_wait(barrier, 3)

    rdmas = []
    for j, nbr in enumerate(peers):
        for i, (src, dst) in enumerate(
                ((acc_ref, r_acc), (m_ref, r_m), (l_ref, r_l))):
            rdma = pltpu.make_async_remote_copy(
                src_ref=src, dst_ref=dst.at[j],
                send_sem=send_sems.at[3 * j + i],
                recv_sem=recv_sems.at[3 * j + i],
                device_id=nbr, device_id_type=MESH)
            rdma.start()
            rdmas.append(rdma)
    for rdma in rdmas:
        rdma.wait()

    m = m_ref[...]
    mt = jnp.maximum(jnp.maximum(m, r_m[0]), jnp.maximum(r_m[1], r_m[2]))
    w = jnp.exp(m - mt)
    lt = w * l_ref[...]
    ot = w * acc_ref[...]
    for j in range(3):
        w = jnp.exp(r_m[j] - mt)
        lt = lt + w * r_l[j]
        ot = ot + w * r_acc[j]
    o = ot / lt
    for h in range(H):
        out_ref[:, 0, h, :] = o[h]


def _combine(acc, m, l):
    return pl.pallas_call(
        _combine_body,
        in_specs=[pl.BlockSpec(memory_space=pltpu.VMEM)] * 3,
        out_specs=pl.BlockSpec(memory_space=pltpu.VMEM),
        out_shape=jax.ShapeDtypeStruct((B, 1, H, D), jnp.float32),
        scratch_shapes=[
            pltpu.VMEM((3, H, B, D), jnp.float32),
            pltpu.VMEM((3, H, B, 1), jnp.float32),
            pltpu.VMEM((3, H, B, 1), jnp.float32),
            pltpu.SemaphoreType.DMA((9,)),
            pltpu.SemaphoreType.DMA((9,)),
        ],
        compiler_params=pltpu.CompilerParams(collective_id=0),
    )(acc, m, l)


def kernel(Q, K, V, bt, lens):
    my_x = lax.axis_index("x")
    q = jnp.transpose(Q.reshape(B, H, D) * SCALE, (1, 0, 2)).astype(
        jnp.bfloat16)
    k = K.reshape(P_LOCAL * BS, H, D)
    v = V.reshape(P_LOCAL * BS, H, D)
    lens2 = lens.reshape(B, 1)
    xarr = jnp.full((1,), my_x * N_KB, jnp.int32)
    acc, m, l = _partial(xarr, q, k, v, bt, lens2)
    return _combine(acc, m, l)


# device time: 53150 ns/iter; 2.7765x vs baseline; 1.6168x over previous
import jax
import jax.numpy as jnp
from jax import lax
from jax.experimental import pallas as pl
from jax.experimental.pallas import tpu as pltpu

B = 32
H = 16
D = 128
BS = 32
NB = 256
P_LOCAL = 256
P_DEV = 128
KB_PAGES = 32
KB_TOK = KB_PAGES * BS
N_KB = P_DEV // KB_PAGES
G = 4
HG = H // G
GB = HG * B
GD = HG * D
NEG = -1e30
SCALE = D ** -0.5
MESH = pl.DeviceIdType.MESH

TOTAL_STEPS = G * N_KB


def _attn_body(xref, qbd_ref, k_hbm, v_hbm, bt_ref, lens_ref,
               acc_ref, m_ref, l_ref,
               bias_ref, kbuf, vbuf, m_sc, l_sc, dsems):
    step = pl.program_id(0)
    kb = step % N_KB
    my_y = lax.axis_index("y")

    def dma_descs(s, slot):
        g = s // N_KB
        row0 = (xref[0] + s % N_KB) * KB_TOK
        descs = []
        for t, (hbm, buf) in enumerate(((k_hbm, kbuf), (v_hbm, vbuf))):
            for a in range(HG):
                descs.append(pltpu.make_async_copy(
                    hbm.at[pl.ds(row0, KB_TOK), g * HG + a],
                    buf.at[slot, :, pl.ds(a * D, D)],
                    dsems.at[slot, t * HG + a]))
        return descs

    @pl.when(step == 0)
    def _prologue():
        for d in dma_descs(0, 0):
            d.start()

    @pl.when(step + 1 < TOTAL_STEPS)
    def _prefetch():
        for d in dma_descs(step + 1, (step + 1) % 2):
            d.start()

    @pl.when(step == 0)
    def _bias():
        bt = bt_ref[...]
        lens = lens_ref[...]
        slot = lax.broadcasted_iota(jnp.int32, (1, 1, NB), 2)
        valid = slot < lens[None, :, :]
        base = my_y * P_LOCAL + xref[0] * KB_PAGES
        CH = 32
        for c in range(P_DEV // CH):
            pages = base + c * CH + lax.broadcasted_iota(
                jnp.int32, (CH, 1, 1), 0)
            eq = bt[None, :, :] == pages
            cnt = jnp.sum(jnp.where(eq & valid, 1.0, 0.0), axis=2)
            bias_ref[c * CH:(c + 1) * CH, :] = jnp.where(
                cnt > 0.5, jnp.log(cnt), NEG)

    @pl.when(kb == 0)
    def _init():
        m_sc[...] = jnp.full((GB, 1), NEG, jnp.float32)
        l_sc[...] = jnp.zeros((GB, 1), jnp.float32)
        acc_ref[...] = jnp.zeros((HG, B, D), jnp.float32)

    bias_blk = bias_ref[pl.ds(kb * KB_PAGES, KB_PAGES), :]
    rows = lax.broadcasted_iota(jnp.int32, (KB_PAGES, KB_TOK), 0)
    cols = lax.broadcasted_iota(jnp.int32, (KB_PAGES, KB_TOK), 1)
    expand = jnp.where(cols // BS == rows, 1.0, 0.0)
    bias_tok = lax.dot_general(
        bias_blk, expand, (((0,), (0,)), ((), ())),
        preferred_element_type=jnp.float32)
    bias_g = jnp.tile(bias_tok, (HG, 1))

    slot = step % 2
    for d in dma_descs(step, slot):
        d.wait()

    kg = kbuf[slot].astype(jnp.bfloat16)
    vg = vbuf[slot].astype(jnp.bfloat16)
    s = lax.dot_general(
        qbd_ref[0], kg, (((1,), (1,)), ((), ())),
        preferred_element_type=jnp.float32)
    s = s + bias_g
    m_old = m_sc[...]
    m_new = jnp.maximum(m_old, jnp.max(s, axis=1, keepdims=True))
    p = jnp.exp(s - m_new)
    corr = jnp.exp(m_old - m_new)
    m_sc[...] = m_new
    l_sc[...] = l_sc[...] * corr + jnp.sum(p, axis=1, keepdims=True)
    o = lax.dot_general(
        p.astype(jnp.bfloat16), vg, (((1,), (0,)), ((), ())),
        preferred_element_type=jnp.float32)
    for a in range(HG):
        acc_ref[a] = (acc_ref[a] * corr[a * B:(a + 1) * B]
                      + o[a * B:(a + 1) * B, a * D:(a + 1) * D])

    @pl.when(kb == N_KB - 1)
    def _finish():
        for a in range(HG):
            m_ref[a] = m_sc[a * B:(a + 1) * B]
            l_ref[a] = l_sc[a * B:(a + 1) * B]


def _partial(xarr, qbd, k, v, bt, lens2):
    grid_spec = pltpu.PrefetchScalarGridSpec(
        num_scalar_prefetch=1,
        grid=(TOTAL_STEPS,),
        in_specs=[
            pl.BlockSpec((1, GB, GD), lambda i, xr: (i // N_KB, 0, 0)),
            pl.BlockSpec(memory_space=pl.ANY),
            pl.BlockSpec(memory_space=pl.ANY),
            pl.BlockSpec((B, NB), lambda i, xr: (0, 0)),
            pl.BlockSpec((B, 1), lambda i, xr: (0, 0)),
        ],
        out_specs=[
            pl.BlockSpec((HG, B, D), lambda i, xr: (i // N_KB, 0, 0)),
            pl.BlockSpec((HG, B, 1), lambda i, xr: (i // N_KB, 0, 0)),
            pl.BlockSpec((HG, B, 1), lambda i, xr: (i // N_KB, 0, 0)),
        ],
        scratch_shapes=[
            pltpu.VMEM((P_DEV, B), jnp.float32),
            pltpu.VMEM((2, KB_TOK, GD), jnp.float32),
            pltpu.VMEM((2, KB_TOK, GD), jnp.float32),
            pltpu.VMEM((GB, 1), jnp.float32),
            pltpu.VMEM((GB, 1), jnp.float32),
            pltpu.SemaphoreType.DMA((2, 2 * HG)),
        ],
    )
    return pl.pallas_call(
        _attn_body,
        grid_spec=grid_spec,
        out_shape=[
            jax.ShapeDtypeStruct((H, B, D), jnp.float32),
            jax.ShapeDtypeStruct((H, B, 1), jnp.float32),
            jax.ShapeDtypeStruct((H, B, 1), jnp.float32),
        ],
        compiler_params=pltpu.CompilerParams(
            dimension_semantics=("arbitrary",)),
    )(xarr, qbd, k, v, bt, lens2)


def _combine_body(acc_ref, m_ref, l_ref, out_ref,
                  s_acc, r_acc, r_m, r_l, send_sems, recv_sems):
    my_x = lax.axis_index("x")
    my_y = lax.axis_index("y")
    peers = (
        (my_x, 1 - my_y),
        (1 - my_x, my_y),
        (1 - my_x, 1 - my_y),
    )

    s_acc[...] = acc_ref[...].astype(jnp.bfloat16)

    barrier = pltpu.get_barrier_semaphore()
    for nbr in peers:
        pl.semaphore_signal(barrier, inc=1, device_id=nbr,
                            device_id_type=MESH)
    pl.semaphore_wait(barrier, 3)

    rdmas = []
    for j, nbr in enumerate(peers):
        for i, (src, dst) in enumerate(
                ((s_acc, r_acc), (m_ref, r_m), (l_ref, r_l))):
            rdma = pltpu.make_async_remote_copy(
                src_ref=src, dst_ref=dst.at[j],
                send_sem=send_sems.at[3 * j + i],
                recv_sem=recv_sems.at[3 * j + i],
                device_id=nbr, device_id_type=MESH)
            rdma.start()
            rdmas.append(rdma)
    for rdma in rdmas:
        rdma.wait()

    m = m_ref[...]
    mt = jnp.maximum(jnp.maximum(m, r_m[0]), jnp.maximum(r_m[1], r_m[2]))
    w = jnp.exp(m - mt)
    lt = w * l_ref[...]
    ot = w * acc_ref[...]
    for j in range(3):
        w = jnp.exp(r_m[j] - mt)
        lt = lt + w * r_l[j]
        ot = ot + w * r_acc[j].astype(jnp.float32)
    out_ref[...] = ot / lt


def _combine(acc, m, l):
    return pl.pallas_call(
        _combine_body,
        in_specs=[pl.BlockSpec(memory_space=pltpu.VMEM)] * 3,
        out_specs=pl.BlockSpec(memory_space=pltpu.VMEM),
        out_shape=jax.ShapeDtypeStruct((H, B, D), jnp.float32),
        scratch_shapes=[
            pltpu.VMEM((H, B, D), jnp.bfloat16),
            pltpu.VMEM((3, H, B, D), jnp.bfloat16),
            pltpu.VMEM((3, H, B, 1), jnp.float32),
            pltpu.VMEM((3, H, B, 1), jnp.float32),
            pltpu.SemaphoreType.DMA((9,)),
            pltpu.SemaphoreType.DMA((9,)),
        ],
        compiler_params=pltpu.CompilerParams(collective_id=0),
    )(acc, m, l)


def kernel(Q, K, V, bt, lens):
    my_x = lax.axis_index("x")
    q = jnp.transpose(Q.reshape(B, H, D) * SCALE, (1, 0, 2))
    qg = q.reshape(G, HG, B, D)
    eye = jnp.eye(HG, dtype=q.dtype)
    qbd = (qg[:, :, :, None, :] * eye[None, :, None, :, None]).reshape(
        G, GB, GD).astype(jnp.bfloat16)
    k = K.reshape(P_LOCAL * BS, H, D)
    v = V.reshape(P_LOCAL * BS, H, D)
    lens2 = lens.reshape(B, 1)
    xarr = jnp.full((1,), my_x * N_KB, jnp.int32)
    acc, m, l = _partial(xarr, qbd, k, v, bt, lens2)
    o = _combine(acc, m, l)
    return jnp.transpose(o, (1, 0, 2)).reshape(B, 1, H, D)


# device time: 51163 ns/iter; 2.8843x vs baseline; 1.0388x over previous
import jax
import jax.numpy as jnp
from jax import lax
from jax.experimental import pallas as pl
from jax.experimental.pallas import tpu as pltpu

B = 32
H = 16
D = 128
BS = 32
NB = 256
P_LOCAL = 256
P_DEV = 128
KB_PAGES = 32
KB_TOK = KB_PAGES * BS
N_KB = P_DEV // KB_PAGES
G = 4
HG = H // G
GB = HG * B
GD = HG * D
NEG = -1e30
SCALE = D ** -0.5
MESH = pl.DeviceIdType.MESH

TOTAL_STEPS = G * N_KB


def _attn_body(xref, qbd_ref, k_hbm, v_hbm, bt_ref, lens_ref, out_ref,
               bias_ref, kbuf, vbuf, m_sc, l_sc,
               acc_sc, m_all, l_all, s_acc, ml_send,
               r_acc, r_ml, dsems,
               asend, arecv, mlsend, mlrecv):
    step = pl.program_id(0)
    kb = step % N_KB
    my_x = lax.axis_index("x")
    my_y = lax.axis_index("y")
    peers = (
        (my_x, 1 - my_y),
        (1 - my_x, my_y),
        (1 - my_x, 1 - my_y),
    )

    def dma_descs(s, slot):
        g = s // N_KB
        row0 = (xref[0] + s % N_KB) * KB_TOK
        descs = []
        for t, (hbm, buf) in enumerate(((k_hbm, kbuf), (v_hbm, vbuf))):
            for a in range(HG):
                descs.append(pltpu.make_async_copy(
                    hbm.at[pl.ds(row0, KB_TOK), g * HG + a],
                    buf.at[slot, :, pl.ds(a * D, D)],
                    dsems.at[slot, t * HG + a]))
        return descs

    def acc_rdma(g, j):
        return pltpu.make_async_remote_copy(
            src_ref=s_acc.at[pl.ds(g * HG, HG)],
            dst_ref=r_acc.at[j, pl.ds(g * HG, HG)],
            send_sem=asend.at[g, j], recv_sem=arecv.at[g, j],
            device_id=peers[j], device_id_type=MESH)

    def ml_rdma(j):
        return pltpu.make_async_remote_copy(
            src_ref=ml_send, dst_ref=r_ml.at[j],
            send_sem=mlsend.at[j], recv_sem=mlrecv.at[j],
            device_id=peers[j], device_id_type=MESH)

    @pl.when(step == 0)
    def _prologue():
        for d in dma_descs(0, 0):
            d.start()
        barrier = pltpu.get_barrier_semaphore()
        for nbr in peers:
            pl.semaphore_signal(barrier, inc=1, device_id=nbr,
                                device_id_type=MESH)
        pl.semaphore_wait(barrier, 3)

    @pl.when(step + 1 < TOTAL_STEPS)
    def _prefetch():
        for d in dma_descs(step + 1, (step + 1) % 2):
            d.start()

    @pl.when(step == 0)
    def _bias():
        bt = bt_ref[...]
        lens = lens_ref[...]
        slot = lax.broadcasted_iota(jnp.int32, (1, 1, NB), 2)
        valid = slot < lens[None, :, :]
        base = my_y * P_LOCAL + xref[0] * KB_PAGES
        CH = 32
        for c in range(P_DEV // CH):
            pages = base + c * CH + lax.broadcasted_iota(
                jnp.int32, (CH, 1, 1), 0)
            eq = bt[None, :, :] == pages
            cnt = jnp.sum(jnp.where(eq & valid, 1.0, 0.0), axis=2)
            bias_ref[c * CH:(c + 1) * CH, :] = jnp.where(
                cnt > 0.5, jnp.log(cnt), NEG)

    @pl.when(kb == 0)
    def _init():
        m_sc[...] = jnp.full((GB, 1), NEG, jnp.float32)
        l_sc[...] = jnp.zeros((GB, 1), jnp.float32)

    bias_blk = bias_ref[pl.ds(kb * KB_PAGES, KB_PAGES), :]
    rows = lax.broadcasted_iota(jnp.int32, (KB_PAGES, KB_TOK), 0)
    cols = lax.broadcasted_iota(jnp.int32, (KB_PAGES, KB_TOK), 1)
    expand = jnp.where(cols // BS == rows, 1.0, 0.0)
    bias_tok = lax.dot_general(
        bias_blk, expand, (((0,), (0,)), ((), ())),
        preferred_element_type=jnp.float32)
    bias_g = jnp.tile(bias_tok, (HG, 1))

    slot = step % 2
    for d in dma_descs(step, slot):
        d.wait()

    g_dyn = step // N_KB
    kg = kbuf[slot].astype(jnp.bfloat16)
    vg = vbuf[slot].astype(jnp.bfloat16)
    s = lax.dot_general(
        qbd_ref[0], kg, (((1,), (1,)), ((), ())),
        preferred_element_type=jnp.float32)
    s = s + bias_g
    m_old = m_sc[...]
    m_new = jnp.maximum(m_old, jnp.max(s, axis=1, keepdims=True))
    p = jnp.exp(s - m_new)
    corr = jnp.exp(m_old - m_new)
    m_sc[...] = m_new
    l_sc[...] = l_sc[...] * corr + jnp.sum(p, axis=1, keepdims=True)
    o = lax.dot_general(
        p.astype(jnp.bfloat16), vg, (((1,), (0,)), ((), ())),
        preferred_element_type=jnp.float32)

    @pl.when(kb == 0)
    def _acc_first():
        for a in range(HG):
            acc_sc[g_dyn * HG + a] = o[a * B:(a + 1) * B,
                                       a * D:(a + 1) * D]

    @pl.when(kb != 0)
    def _acc_rest():
        for a in range(HG):
            acc_sc[g_dyn * HG + a] = (
                acc_sc[g_dyn * HG + a] * corr[a * B:(a + 1) * B]
                + o[a * B:(a + 1) * B, a * D:(a + 1) * D])

    @pl.when(kb == N_KB - 1)
    def _group_done():
        m_all[pl.ds(g_dyn * HG, HG)] = m_sc[...].reshape(HG, B, 1)
        l_all[pl.ds(g_dyn * HG, HG)] = l_sc[...].reshape(HG, B, 1)
        rows_g = pl.ds(g_dyn * HG, HG)
        s_acc[rows_g] = acc_sc[rows_g].astype(jnp.bfloat16)
        for j in range(3):
            pltpu.make_async_remote_copy(
                src_ref=s_acc.at[rows_g],
                dst_ref=r_acc.at[j, rows_g],
                send_sem=asend.at[g_dyn, j], recv_sem=arecv.at[g_dyn, j],
                device_id=peers[j], device_id_type=MESH).start()

    @pl.when(step == TOTAL_STEPS - 1)
    def _finish():
        ml_send[0:H] = m_all[...]
        ml_send[H:2 * H] = l_all[...]
        for j in range(3):
            ml_rdma(j).start()
        for g in range(G):
            for j in range(3):
                acc_rdma(g, j).wait()
        for j in range(3):
            ml_rdma(j).wait()

        m = m_all[...]
        r_m = [r_ml[j, 0:H] for j in range(3)]
        r_l = [r_ml[j, H:2 * H] for j in range(3)]
        mt = jnp.maximum(jnp.maximum(m, r_m[0]),
                         jnp.maximum(r_m[1], r_m[2]))
        w = jnp.exp(m - mt)
        lt = w * l_all[...]
        ot = w * acc_sc[...]
        for j in range(3):
            w = jnp.exp(r_m[j] - mt)
            lt = lt + w * r_l[j]
            ot = ot + w * r_acc[j].astype(jnp.float32)
        out_ref[...] = ot / lt


def _attention(xarr, qbd, k, v, bt, lens2):
    grid_spec = pltpu.PrefetchScalarGridSpec(
        num_scalar_prefetch=1,
        grid=(TOTAL_STEPS,),
        in_specs=[
            pl.BlockSpec((1, GB, GD), lambda i, xr: (i // N_KB, 0, 0)),
            pl.BlockSpec(memory_space=pl.ANY),
            pl.BlockSpec(memory_space=pl.ANY),
            pl.BlockSpec((B, NB), lambda i, xr: (0, 0)),
            pl.BlockSpec((B, 1), lambda i, xr: (0, 0)),
        ],
        out_specs=[
            pl.BlockSpec((H, B, D), lambda i, xr: (0, 0, 0)),
        ],
        scratch_shapes=[
            pltpu.VMEM((P_DEV, B), jnp.float32),
            pltpu.VMEM((2, KB_TOK, GD), jnp.float32),
            pltpu.VMEM((2, KB_TOK, GD), jnp.float32),
            pltpu.VMEM((GB, 1), jnp.float32),
            pltpu.VMEM((GB, 1), jnp.float32),
            pltpu.VMEM((H, B, D), jnp.float32),
            pltpu.VMEM((H, B, 1), jnp.float32),
            pltpu.VMEM((H, B, 1), jnp.float32),
            pltpu.VMEM((H, B, D), jnp.bfloat16),
            pltpu.VMEM((2 * H, B, 1), jnp.float32),
            pltpu.VMEM((3, H, B, D), jnp.bfloat16),
            pltpu.VMEM((3, 2 * H, B, 1), jnp.float32),
            pltpu.SemaphoreType.DMA((2, 2 * HG)),
            pltpu.SemaphoreType.DMA((G, 3)),
            pltpu.SemaphoreType.DMA((G, 3)),
            pltpu.SemaphoreType.DMA((3,)),
            pltpu.SemaphoreType.DMA((3,)),
        ],
    )
    return pl.pallas_call(
        _attn_body,
        grid_spec=grid_spec,
        out_shape=[
            jax.ShapeDtypeStruct((H, B, D), jnp.float32),
        ],
        compiler_params=pltpu.CompilerParams(
            dimension_semantics=("arbitrary",),
            collective_id=0),
    )(xarr, qbd, k, v, bt, lens2)


def kernel(Q, K, V, bt, lens):
    my_x = lax.axis_index("x")
    q = jnp.transpose(Q.reshape(B, H, D) * SCALE, (1, 0, 2))
    qg = q.reshape(G, HG, B, D)
    eye = jnp.eye(HG, dtype=q.dtype)
    qbd = (qg[:, :, :, None, :] * eye[None, :, None, :, None]).reshape(
        G, GB, GD).astype(jnp.bfloat16)
    k = K.reshape(P_LOCAL * BS, H, D)
    v = V.reshape(P_LOCAL * BS, H, D)
    lens2 = lens.reshape(B, 1)
    xarr = jnp.full((1,), my_x * N_KB, jnp.int32)
    (o,) = _attention(xarr, qbd, k, v, bt, lens2)
    return jnp.transpose(o, (1, 0, 2)).reshape(B, 1, H, D)


# device time: 45985 ns/iter; 3.2091x vs baseline; 1.1126x over previous
import jax
import jax.numpy as jnp
from jax import lax
from jax.experimental import pallas as pl
from jax.experimental.pallas import tpu as pltpu

B = 32
H = 16
D = 128
BS = 32
NB = 256
P_LOCAL = 256
P_DEV = 128
KB_PAGES = 64
KB_TOK = KB_PAGES * BS
N_KB = P_DEV // KB_PAGES
G = 4
HG = H // G
GB = HG * B
GD = HG * D
NEG = -1e30
SCALE = D ** -0.5
MESH = pl.DeviceIdType.MESH

TOTAL_STEPS = G * N_KB


def _attn_body(xref, qbd_ref, k_hbm, v_hbm, bt_ref, lens_ref, out_ref,
               bias_ref, kbuf, vbuf, m_sc, l_sc,
               acc_cur, m_all, l_all, s_acc, ml_send,
               r_acc, r_ml, dsems,
               asend, arecv, mlsend, mlrecv):
    step = pl.program_id(0)
    kb = step % N_KB
    my_x = lax.axis_index("x")
    my_y = lax.axis_index("y")
    peers = (
        (my_x, 1 - my_y),
        (1 - my_x, my_y),
        (1 - my_x, 1 - my_y),
    )

    def dma_descs(s, slot):
        g = s // N_KB
        row0 = (xref[0] + s % N_KB) * KB_TOK
        descs = []
        for t, (hbm, buf) in enumerate(((k_hbm, kbuf), (v_hbm, vbuf))):
            for a in range(HG):
                descs.append(pltpu.make_async_copy(
                    hbm.at[pl.ds(row0, KB_TOK), g * HG + a],
                    buf.at[slot, :, pl.ds(a * D, D)],
                    dsems.at[slot, t * HG + a]))
        return descs

    def acc_rdma(g, j):
        return pltpu.make_async_remote_copy(
            src_ref=s_acc.at[pl.ds(g * HG, HG)],
            dst_ref=r_acc.at[j, pl.ds(g * HG, HG)],
            send_sem=asend.at[g, j], recv_sem=arecv.at[g, j],
            device_id=peers[j], device_id_type=MESH)

    def ml_rdma(j):
        return pltpu.make_async_remote_copy(
            src_ref=ml_send, dst_ref=r_ml.at[j],
            send_sem=mlsend.at[j], recv_sem=mlrecv.at[j],
            device_id=peers[j], device_id_type=MESH)

    @pl.when(step == 0)
    def _prologue():
        for d in dma_descs(0, 0):
            d.start()
        barrier = pltpu.get_barrier_semaphore()
        for nbr in peers:
            pl.semaphore_signal(barrier, inc=1, device_id=nbr,
                                device_id_type=MESH)
        pl.semaphore_wait(barrier, 3)

    @pl.when(step + 1 < TOTAL_STEPS)
    def _prefetch():
        for d in dma_descs(step + 1, (step + 1) % 2):
            d.start()

    @pl.when(step == 0)
    def _bias():
        bt = bt_ref[...]
        lens = lens_ref[...]
        slot = lax.broadcasted_iota(jnp.int32, (1, 1, NB), 2)
        valid = slot < lens[None, :, :]
        base = my_y * P_LOCAL + xref[0] * KB_PAGES
        CH = 32
        for c in range(P_DEV // CH):
            pages = base + c * CH + lax.broadcasted_iota(
                jnp.int32, (CH, 1, 1), 0)
            eq = bt[None, :, :] == pages
            cnt = jnp.sum(jnp.where(eq & valid, 1.0, 0.0), axis=2)
            bias_ref[c * CH:(c + 1) * CH, :] = jnp.where(
                cnt > 0.5, jnp.log(cnt), NEG)

    @pl.when(kb == 0)
    def _init():
        m_sc[...] = jnp.full((GB, 1), NEG, jnp.float32)
        l_sc[...] = jnp.zeros((GB, 1), jnp.float32)

    bias_blk = bias_ref[pl.ds(kb * KB_PAGES, KB_PAGES), :]
    rows = lax.broadcasted_iota(jnp.int32, (KB_PAGES, KB_TOK), 0)
    cols = lax.broadcasted_iota(jnp.int32, (KB_PAGES, KB_TOK), 1)
    expand = jnp.where(cols // BS == rows, 1.0, 0.0)
    bias_tok = lax.dot_general(
        bias_blk, expand, (((0,), (0,)), ((), ())),
        preferred_element_type=jnp.float32)
    bias_g = jnp.tile(bias_tok, (HG, 1))

    slot = step % 2
    for d in dma_descs(step, slot):
        d.wait()

    g_dyn = step // N_KB
    kg = kbuf[slot].astype(jnp.bfloat16)
    vg = vbuf[slot].astype(jnp.bfloat16)
    s = lax.dot_general(
        qbd_ref[0], kg, (((1,), (1,)), ((), ())),
        preferred_element_type=jnp.float32)
    s = s + bias_g
    m_old = m_sc[...]
    m_new = jnp.maximum(m_old, jnp.max(s, axis=1, keepdims=True))
    p = jnp.exp(s - m_new)
    corr = jnp.exp(m_old - m_new)
    m_sc[...] = m_new
    l_sc[...] = l_sc[...] * corr + jnp.sum(p, axis=1, keepdims=True)
    o = lax.dot_general(
        p.astype(jnp.bfloat16), vg, (((1,), (0,)), ((), ())),
        preferred_element_type=jnp.float32)

    @pl.when(kb == 0)
    def _acc_first():
        for a in range(HG):
            acc_cur[a] = o[a * B:(a + 1) * B, a * D:(a + 1) * D]

    @pl.when(kb != 0)
    def _acc_rest():
        for a in range(HG):
            acc_cur[a] = (acc_cur[a] * corr[a * B:(a + 1) * B]
                          + o[a * B:(a + 1) * B, a * D:(a + 1) * D])

    @pl.when(kb == N_KB - 1)
    def _group_done():
        m_all[pl.ds(g_dyn * HG, HG)] = m_sc[...].reshape(HG, B, 1)
        l_all[pl.ds(g_dyn * HG, HG)] = l_sc[...].reshape(HG, B, 1)
        rows_g = pl.ds(g_dyn * HG, HG)
        s_acc[rows_g] = acc_cur[...].astype(jnp.bfloat16)
        for j in range(3):
            pltpu.make_async_remote_copy(
                src_ref=s_acc.at[rows_g],
                dst_ref=r_acc.at[j, rows_g],
                send_sem=asend.at[g_dyn, j], recv_sem=arecv.at[g_dyn, j],
                device_id=peers[j], device_id_type=MESH).start()

    @pl.when(step == TOTAL_STEPS - 1)
    def _finish():
        ml_send[0:H] = m_all[...]
        ml_send[H:2 * H] = l_all[...]
        for j in range(3):
            ml_rdma(j).start()
        for g in range(G):
            for j in range(3):
                acc_rdma(g, j).wait()
        for j in range(3):
            ml_rdma(j).wait()

        m = m_all[...]
        r_m = [r_ml[j, 0:H] for j in range(3)]
        r_l = [r_ml[j, H:2 * H] for j in range(3)]
        mt = jnp.maximum(jnp.maximum(m, r_m[0]),
                         jnp.maximum(r_m[1], r_m[2]))
        w = jnp.exp(m - mt)
        lt = w * l_all[...]
        ot = w * s_acc[...].astype(jnp.float32)
        for j in range(3):
            w = jnp.exp(r_m[j] - mt)
            lt = lt + w * r_l[j]
            ot = ot + w * r_acc[j].astype(jnp.float32)
        out_ref[...] = ot / lt


def _attention(xarr, qbd, k, v, bt, lens2):
    grid_spec = pltpu.PrefetchScalarGridSpec(
        num_scalar_prefetch=1,
        grid=(TOTAL_STEPS,),
        in_specs=[
            pl.BlockSpec((1, GB, GD), lambda i, xr: (i // N_KB, 0, 0)),
            pl.BlockSpec(memory_space=pl.ANY),
            pl.BlockSpec(memory_space=pl.ANY),
            pl.BlockSpec((B, NB), lambda i, xr: (0, 0)),
            pl.BlockSpec((B, 1), lambda i, xr: (0, 0)),
        ],
        out_specs=[
            pl.BlockSpec((H, B, D), lambda i, xr: (0, 0, 0)),
        ],
        scratch_shapes=[
            pltpu.VMEM((P_DEV, B), jnp.float32),
            pltpu.VMEM((2, KB_TOK, GD), jnp.float32),
            pltpu.VMEM((2, KB_TOK, GD), jnp.float32),
            pltpu.VMEM((GB, 1), jnp.float32),
            pltpu.VMEM((GB, 1), jnp.float32),
            pltpu.VMEM((HG, B, D), jnp.float32),
            pltpu.VMEM((H, B, 1), jnp.float32),
            pltpu.VMEM((H, B, 1), jnp.float32),
            pltpu.VMEM((H, B, D), jnp.bfloat16),
            pltpu.VMEM((2 * H, B, 1), jnp.float32),
            pltpu.VMEM((3, H, B, D), jnp.bfloat16),
            pltpu.VMEM((3, 2 * H, B, 1), jnp.float32),
            pltpu.SemaphoreType.DMA((2, 2 * HG)),
            pltpu.SemaphoreType.DMA((G, 3)),
            pltpu.SemaphoreType.DMA((G, 3)),
            pltpu.SemaphoreType.DMA((3,)),
            pltpu.SemaphoreType.DMA((3,)),
        ],
    )
    return pl.pallas_call(
        _attn_body,
        grid_spec=grid_spec,
        out_shape=[
            jax.ShapeDtypeStruct((H, B, D), jnp.float32),
        ],
        compiler_params=pltpu.CompilerParams(
            dimension_semantics=("arbitrary",),
            collective_id=0),
    )(xarr, qbd, k, v, bt, lens2)


def kernel(Q, K, V, bt, lens):
    my_x = lax.axis_index("x")
    q = jnp.transpose(Q.reshape(B, H, D) * SCALE, (1, 0, 2))
    qg = q.reshape(G, HG, B, D)
    eye = jnp.eye(HG, dtype=q.dtype)
    qbd = (qg[:, :, :, None, :] * eye[None, :, None, :, None]).reshape(
        G, GB, GD).astype(jnp.bfloat16)
    k = K.reshape(P_LOCAL * BS, H, D)
    v = V.reshape(P_LOCAL * BS, H, D)
    lens2 = lens.reshape(B, 1)
    xarr = jnp.full((1,), my_x * N_KB, jnp.int32)
    (o,) = _attention(xarr, qbd, k, v, bt, lens2)
    return jnp.transpose(o, (1, 0, 2)).reshape(B, 1, H, D)
